# baseline (speedup 1.0000x reference)
import functools

import jax
import jax.numpy as jnp
from jax import lax
from jax.experimental import pallas as pl
from jax.experimental.pallas import tpu as pltpu
from jax.experimental.pallas import tpu_sc as plsc

BATCH = 16384
LATENT = 32
PAIRS = LATENT // 2
NUM_CORES = 2
NUM_SUBCORES = 16
NUM_WORKERS = NUM_CORES * NUM_SUBCORES
ROWS_PER_WORKER = BATCH // NUM_WORKERS
IDX_CHUNK = 128
NUM_CHUNKS = ROWS_PER_WORKER // IDX_CHUNK
LANES = 16
GROUPS = ROWS_PER_WORKER // LANES
HI_MASK = jnp.int32(-65536)


def _gmf_body(uidx_hbm, iidx_hbm, utab_hbm, itab_hbm, wb_hbm, out_hbm,
              uidx_v, iidx_v, urows_v, irows_v, wb_v, out_v, sem):
    wid = lax.axis_index("s") * NUM_CORES + lax.axis_index("c")

    pltpu.sync_copy(uidx_hbm.at[wid], uidx_v)
    pltpu.sync_copy(iidx_hbm.at[wid], iidx_v)
    pltpu.sync_copy(wb_hbm, wb_v)

    copies = []
    for j in range(NUM_CHUNKS):
        dst = pl.ds(j * IDX_CHUNK, IDX_CHUNK)
        copies.append(pltpu.async_copy(utab_hbm.at[uidx_v.at[j]],
                                       urows_v.at[dst], sem))
        copies.append(pltpu.async_copy(itab_hbm.at[iidx_v.at[j]],
                                       irows_v.at[dst], sem))
    for c in copies:
        c.wait()

    lane_ids = lax.iota(jnp.int32, LANES)
    w_lo = wb_v[pl.ds(0, LANES)]
    w_hi = wb_v[pl.ds(LANES, LANES)]
    bias = wb_v[pl.ds(2 * LANES, LANES)][0]

    def expand(packed):
        even = plsc.bitcast(packed << 16, jnp.float32)
        odd = plsc.bitcast(packed & HI_MASK, jnp.float32)
        return even, odd

    def group_body(g, carry):
        rows = g * LANES + lane_ids
        acc = jnp.zeros((LANES,), jnp.float32) + bias
        for q in range(PAIRS):
            dims = jnp.full((LANES,), q, dtype=jnp.int32)
            u_pk = plsc.load_gather(urows_v, [rows, dims])
            v_pk = plsc.load_gather(irows_v, [rows, dims])
            u_e, u_o = expand(u_pk)
            v_e, v_o = expand(v_pk)
            w_e = w_lo[2 * q] if q < 8 else w_hi[2 * q - 16]
            w_o = w_lo[2 * q + 1] if q < 8 else w_hi[2 * q - 15]
            acc = acc + u_e * v_e * w_e + u_o * v_o * w_o
        out_v[pl.ds(g * LANES, LANES)] = acc
        return carry

    lax.fori_loop(0, GROUPS, group_body, 0)

    pltpu.sync_copy(out_v, out_hbm.at[pl.ds(wid * ROWS_PER_WORKER,
                                            ROWS_PER_WORKER)])


def _pack_bf16_pairs(table):
    b16 = lax.bitcast_convert_type(table.astype(jnp.bfloat16), jnp.uint16)
    lo = b16[:, 0::2].astype(jnp.uint32)
    hi = b16[:, 1::2].astype(jnp.uint32)
    return lax.bitcast_convert_type(lo | (hi << 16), jnp.int32)


@jax.jit
def kernel(user_indices, item_indices, user_table, item_table,
           affine_W, affine_b):
    ui = user_indices.astype(jnp.int32).reshape(NUM_WORKERS, NUM_CHUNKS,
                                                IDX_CHUNK)
    ii = item_indices.astype(jnp.int32).reshape(NUM_WORKERS, NUM_CHUNKS,
                                                IDX_CHUNK)
    upk = _pack_bf16_pairs(user_table)
    ipk = _pack_bf16_pairs(item_table)
    wb = jnp.concatenate([affine_W.reshape(LATENT), affine_b,
                          jnp.zeros((15,), jnp.float32)])

    mesh = plsc.VectorSubcoreMesh(core_axis_name="c", subcore_axis_name="s")
    run = functools.partial(
        pl.kernel,
        out_type=jax.ShapeDtypeStruct((BATCH,), jnp.float32),
        mesh=mesh,
        compiler_params=pltpu.CompilerParams(needs_layout_passes=False,
                                             use_tc_tiling_on_sc=False),
        scratch_types=[
            pltpu.VMEM((NUM_CHUNKS, IDX_CHUNK), jnp.int32),
            pltpu.VMEM((NUM_CHUNKS, IDX_CHUNK), jnp.int32),
            pltpu.VMEM((ROWS_PER_WORKER, PAIRS), jnp.int32),
            pltpu.VMEM((ROWS_PER_WORKER, PAIRS), jnp.int32),
            pltpu.VMEM((LATENT + 16,), jnp.float32),
            pltpu.VMEM((ROWS_PER_WORKER,), jnp.float32),
            pltpu.SemaphoreType.DMA,
        ],
    )(_gmf_body)
    scores = run(ui, ii, upk, ipk, wb)
    return scores.reshape(BATCH, 1)


# speedup vs baseline: 7.4776x; 7.4776x over previous
import functools

import jax
import jax.numpy as jnp
from jax import lax
from jax.experimental import pallas as pl
from jax.experimental.pallas import tpu as pltpu
from jax.experimental.pallas import tpu_sc as plsc

BATCH = 16384
LATENT = 32
PAIRS = LATENT // 2
NUM_CORES = 2
NUM_SUBCORES = 16
NUM_WORKERS = NUM_CORES * NUM_SUBCORES
ROWS_PER_WORKER = BATCH // NUM_WORKERS
IDX_CHUNK = 128
NUM_CHUNKS = ROWS_PER_WORKER // IDX_CHUNK
LANES = 16
GROUPS = ROWS_PER_WORKER // LANES
HI_MASK = jnp.int32(-65536)


def _gmf_body(uidx_hbm, iidx_hbm, utab_hbm, itab_hbm, wb_hbm, out_hbm,
              uidx_v, iidx_v, urows_v, irows_v, wb_v, out_v, sem):
    wid = lax.axis_index("s") * NUM_CORES + lax.axis_index("c")

    pltpu.sync_copy(uidx_hbm.at[wid], uidx_v)
    pltpu.sync_copy(iidx_hbm.at[wid], iidx_v)
    pltpu.sync_copy(wb_hbm, wb_v)

    copies = []
    for j in range(NUM_CHUNKS):
        dst = pl.ds(j * IDX_CHUNK, IDX_CHUNK)
        copies.append(pltpu.async_copy(utab_hbm.at[uidx_v.at[j]],
                                       urows_v.at[dst], sem))
        copies.append(pltpu.async_copy(itab_hbm.at[iidx_v.at[j]],
                                       irows_v.at[dst], sem))
    for c in copies:
        c.wait()

    lane_ids = lax.iota(jnp.int32, LANES)
    w_lo = wb_v[pl.ds(0, LANES)]
    w_hi = wb_v[pl.ds(LANES, LANES)]
    bias = wb_v[pl.ds(2 * LANES, LANES)][0]

    def expand(packed):
        even = plsc.bitcast(packed << 16, jnp.float32)
        odd = plsc.bitcast(packed & HI_MASK, jnp.float32)
        return even, odd

    def group_body(g, carry):
        rows = g * LANES + lane_ids
        acc = jnp.zeros((LANES,), jnp.float32) + bias
        for q in range(PAIRS):
            dims = jnp.full((LANES,), q, dtype=jnp.int32)
            u_pk = plsc.load_gather(urows_v, [rows, dims])
            v_pk = plsc.load_gather(irows_v, [rows, dims])
            u_e, u_o = expand(u_pk)
            v_e, v_o = expand(v_pk)
            w_e = w_lo[2 * q] if q < 8 else w_hi[2 * q - 16]
            w_o = w_lo[2 * q + 1] if q < 8 else w_hi[2 * q - 15]
            acc = acc + u_e * v_e * w_e + u_o * v_o * w_o
        out_v[pl.ds(g * LANES, LANES)] = acc
        return carry

    lax.fori_loop(0, GROUPS, group_body, 0)

    pltpu.sync_copy(out_v, out_hbm.at[pl.ds(wid * ROWS_PER_WORKER,
                                            ROWS_PER_WORKER)])


def _pack_bf16_pairs(table):
    b16 = table.astype(jnp.bfloat16).reshape(table.shape[0], PAIRS, 2)
    return lax.bitcast_convert_type(b16, jnp.int32)


@jax.jit
def kernel(user_indices, item_indices, user_table, item_table,
           affine_W, affine_b):
    ui = user_indices.astype(jnp.int32).reshape(NUM_WORKERS, NUM_CHUNKS,
                                                IDX_CHUNK)
    ii = item_indices.astype(jnp.int32).reshape(NUM_WORKERS, NUM_CHUNKS,
                                                IDX_CHUNK)
    upk = _pack_bf16_pairs(user_table)
    ipk = _pack_bf16_pairs(item_table)
    wb = jnp.concatenate([affine_W.reshape(LATENT), affine_b,
                          jnp.zeros((15,), jnp.float32)])

    mesh = plsc.VectorSubcoreMesh(core_axis_name="c", subcore_axis_name="s")
    run = functools.partial(
        pl.kernel,
        out_type=jax.ShapeDtypeStruct((BATCH,), jnp.float32),
        mesh=mesh,
        compiler_params=pltpu.CompilerParams(needs_layout_passes=False,
                                             use_tc_tiling_on_sc=False),
        scratch_types=[
            pltpu.VMEM((NUM_CHUNKS, IDX_CHUNK), jnp.int32),
            pltpu.VMEM((NUM_CHUNKS, IDX_CHUNK), jnp.int32),
            pltpu.VMEM((ROWS_PER_WORKER, PAIRS), jnp.int32),
            pltpu.VMEM((ROWS_PER_WORKER, PAIRS), jnp.int32),
            pltpu.VMEM((LATENT + 16,), jnp.float32),
            pltpu.VMEM((ROWS_PER_WORKER,), jnp.float32),
            pltpu.SemaphoreType.DMA,
        ],
    )(_gmf_body)
    scores = run(ui, ii, upk, ipk, wb)
    return scores.reshape(BATCH, 1)


# speedup vs baseline: 16.2594x; 2.1744x over previous
import functools

import jax
import jax.numpy as jnp
from jax import lax
from jax.experimental import pallas as pl
from jax.experimental.pallas import tpu as pltpu
from jax.experimental.pallas import tpu_sc as plsc

BATCH = 16384
LATENT = 32
NUM_CORES = 2
NUM_SUBCORES = 16
NUM_WORKERS = NUM_CORES * NUM_SUBCORES
ROWS_PER_WORKER = BATCH // NUM_WORKERS
IDX_CHUNK = 128
NUM_CHUNKS = ROWS_PER_WORKER // IDX_CHUNK
LANES = 16
GROUPS = ROWS_PER_WORKER // LANES


def _gmf_body(uidx_hbm, iidx_hbm, utab_hbm, itab_hbm, wb_hbm, out_hbm,
              uidx_v, iidx_v, urows_v, irows_v, wb_v, out_v, sem):
    wid = lax.axis_index("s") * NUM_CORES + lax.axis_index("c")

    pltpu.sync_copy(uidx_hbm.at[wid], uidx_v)
    pltpu.sync_copy(iidx_hbm.at[wid], iidx_v)
    pltpu.sync_copy(wb_hbm, wb_v)

    copies = []
    for j in range(NUM_CHUNKS):
        dst = pl.ds(j * IDX_CHUNK, IDX_CHUNK)
        copies.append(pltpu.async_copy(utab_hbm.at[uidx_v.at[j]],
                                       urows_v.at[dst], sem))
        copies.append(pltpu.async_copy(itab_hbm.at[iidx_v.at[j]],
                                       irows_v.at[dst], sem))
    for c in copies:
        c.wait()

    lane_ids = lax.iota(jnp.int32, LANES)
    w_lo = wb_v[pl.ds(0, LANES)]
    w_hi = wb_v[pl.ds(LANES, LANES)]
    bias = wb_v[pl.ds(2 * LANES, LANES)][0]

    def group_body(g, carry):
        rows = g * LANES + lane_ids
        acc = jnp.zeros((LANES,), jnp.float32) + bias
        for d in range(LATENT):
            dims = jnp.full((LANES,), d, dtype=jnp.int32)
            u = plsc.load_gather(urows_v, [rows, dims])
            v = plsc.load_gather(irows_v, [rows, dims])
            w = w_lo[d] if d < LANES else w_hi[d - LANES]
            acc = acc + u * v * w
        out_v[pl.ds(g * LANES, LANES)] = acc
        return carry

    lax.fori_loop(0, GROUPS, group_body, 0)

    pltpu.sync_copy(out_v, out_hbm.at[pl.ds(wid * ROWS_PER_WORKER,
                                            ROWS_PER_WORKER)])


@jax.jit
def kernel(user_indices, item_indices, user_table, item_table,
           affine_W, affine_b):
    ui = user_indices.astype(jnp.int32).reshape(NUM_WORKERS, NUM_CHUNKS,
                                                IDX_CHUNK)
    ii = item_indices.astype(jnp.int32).reshape(NUM_WORKERS, NUM_CHUNKS,
                                                IDX_CHUNK)
    wb = jnp.concatenate([affine_W.reshape(LATENT), affine_b,
                          jnp.zeros((15,), jnp.float32)])

    mesh = plsc.VectorSubcoreMesh(core_axis_name="c", subcore_axis_name="s")
    run = functools.partial(
        pl.kernel,
        out_type=jax.ShapeDtypeStruct((BATCH,), jnp.float32),
        mesh=mesh,
        compiler_params=pltpu.CompilerParams(needs_layout_passes=False,
                                             use_tc_tiling_on_sc=False),
        scratch_types=[
            pltpu.VMEM((NUM_CHUNKS, IDX_CHUNK), jnp.int32),
            pltpu.VMEM((NUM_CHUNKS, IDX_CHUNK), jnp.int32),
            pltpu.VMEM((ROWS_PER_WORKER, LATENT), jnp.float32),
            pltpu.VMEM((ROWS_PER_WORKER, LATENT), jnp.float32),
            pltpu.VMEM((LATENT + 16,), jnp.float32),
            pltpu.VMEM((ROWS_PER_WORKER,), jnp.float32),
            pltpu.SemaphoreType.DMA,
        ],
    )(_gmf_body)
    scores = run(ui, ii, user_table, item_table, wb)
    return scores.reshape(BATCH, 1)
